# baseline (device time: 36030 ns/iter reference)
import functools

import jax
import jax.numpy as jnp
from jax import lax
from jax.experimental import pallas as pl
from jax.experimental.pallas import tpu as pltpu

N_DEV = 4
B, SQ, SKV, HQ, DH = 2, 128, 512, 16, 64
H_SH = HQ // N_DEV
KV_SH = SKV // N_DEV
DMODEL = 512
SCALE = 0.125


def kernel(x, Wq, K_ext, V_ext, Wo):
    def body(x_ref, wq_ref, k_ref, v_ref, wo_ref, out_ref,
             kstage, vstage, kg, vg, prec,
             ksend_sems, vsend_sems, psend_sems,
             krecv_sems, vrecv_sems, precv_sems):
        p = lax.axis_index("i")

        bar = pltpu.get_barrier_semaphore()
        for d in range(1, N_DEV):
            pl.semaphore_signal(
                bar, inc=1,
                device_id=(lax.rem(p + d, N_DEV),),
                device_id_type=pl.DeviceIdType.MESH,
            )
        pl.semaphore_wait(bar, N_DEV - 1)

        for hg in range(N_DEV):
            k_slice = k_ref[:, :, hg * H_SH:(hg + 1) * H_SH, :].astype(jnp.bfloat16)
            v_slice = v_ref[:, :, hg * H_SH:(hg + 1) * H_SH, :].astype(jnp.bfloat16)

            @pl.when(hg == p)
            def _(k_slice=k_slice, v_slice=v_slice):
                kg[hg] = k_slice
                vg[hg] = v_slice

            @pl.when(hg != p)
            def _(hg=hg, k_slice=k_slice, v_slice=v_slice):
                kstage[hg] = k_slice
                vstage[hg] = v_slice
                rk = pltpu.make_async_remote_copy(
                    src_ref=kstage.at[hg], dst_ref=kg.at[p],
                    send_sem=ksend_sems.at[hg], recv_sem=krecv_sems.at[p],
                    device_id=(hg,), device_id_type=pl.DeviceIdType.MESH,
                )
                rk.start()
                rv = pltpu.make_async_remote_copy(
                    src_ref=vstage.at[hg], dst_ref=vg.at[p],
                    send_sem=vsend_sems.at[hg], recv_sem=vrecv_sems.at[p],
                    device_id=(hg,), device_id_type=pl.DeviceIdType.MESH,
                )
                rv.start()

        wq = wq_ref[:, :].astype(jnp.bfloat16)
        q_b = []
        for b in range(B):
            qb = lax.dot_general(
                x_ref[b].astype(jnp.bfloat16), wq,
                (((1,), (0,)), ((), ())),
                preferred_element_type=jnp.float32,
            )
            q_b.append(qb.astype(jnp.bfloat16))

        qi = lax.broadcasted_iota(jnp.int32, (SQ, SKV), 0)
        ki = lax.broadcasted_iota(jnp.int32, (SQ, SKV), 1)
        mask = (ki <= qi + 128) | (qi < 32)

        for d in range(1, N_DEV):
            o = lax.rem(p + d, N_DEV)
            rk = pltpu.make_async_remote_copy(
                src_ref=kg.at[o], dst_ref=kg.at[o],
                send_sem=ksend_sems.at[0], recv_sem=krecv_sems.at[o],
                device_id=(o,), device_id_type=pl.DeviceIdType.MESH,
            )
            rk.wait_recv()
            rv = pltpu.make_async_remote_copy(
                src_ref=vg.at[o], dst_ref=vg.at[o],
                send_sem=vsend_sems.at[0], recv_sem=vrecv_sems.at[o],
                device_id=(o,), device_id_type=pl.DeviceIdType.MESH,
            )
            rv.wait_recv()

        wo = wo_ref[:, :].astype(jnp.bfloat16)
        for b in range(B):
            ctx_h = []
            for h in range(H_SH):
                q_bh = q_b[b][:, h * DH:(h + 1) * DH]
                s_chunks = []
                for o in range(N_DEV):
                    k_o = kg[o, b, :, h, :]
                    s_chunks.append(lax.dot_general(
                        q_bh, k_o, (((1,), (1,)), ((), ())),
                        preferred_element_type=jnp.float32,
                    ))
                s = jnp.concatenate(s_chunks, axis=1) * SCALE
                s = jnp.where(mask, s, -1e9)
                m = jnp.max(s, axis=1, keepdims=True)
                e = jnp.exp(s - m)
                w = (e / jnp.sum(e, axis=1, keepdims=True)).astype(jnp.bfloat16)
                v_full = jnp.concatenate(
                    [vg[o, b, :, h, :] for o in range(N_DEV)], axis=0
                )
                ctx_h.append(lax.dot_general(
                    w, v_full, (((1,), (0,)), ((), ())),
                    preferred_element_type=jnp.float32,
                ))
            ctx_b = jnp.concatenate(ctx_h, axis=1).astype(jnp.bfloat16)
            partial_b = lax.dot_general(
                ctx_b, wo, (((1,), (0,)), ((), ())),
                preferred_element_type=jnp.float32,
            )
            prec[p, b] = partial_b.astype(jnp.bfloat16)

        for d in range(1, N_DEV):
            q = lax.rem(p + d, N_DEV)
            rp = pltpu.make_async_remote_copy(
                src_ref=prec.at[p], dst_ref=prec.at[p],
                send_sem=psend_sems.at[q], recv_sem=precv_sems.at[p],
                device_id=(q,), device_id_type=pl.DeviceIdType.MESH,
            )
            rp.start()

        for d in range(1, N_DEV):
            o = lax.rem(p + d, N_DEV)
            rp = pltpu.make_async_remote_copy(
                src_ref=prec.at[o], dst_ref=prec.at[o],
                send_sem=psend_sems.at[0], recv_sem=precv_sems.at[o],
                device_id=(o,), device_id_type=pl.DeviceIdType.MESH,
            )
            rp.wait_recv()

        total = prec[0].astype(jnp.float32)
        for o in range(1, N_DEV):
            total = total + prec[o].astype(jnp.float32)
        out_ref[...] = total

        for hg in range(N_DEV):
            @pl.when(hg != p)
            def _(hg=hg):
                rk = pltpu.make_async_remote_copy(
                    src_ref=kstage.at[hg], dst_ref=kg.at[p],
                    send_sem=ksend_sems.at[hg], recv_sem=krecv_sems.at[p],
                    device_id=(hg,), device_id_type=pl.DeviceIdType.MESH,
                )
                rk.wait_send()
                rv = pltpu.make_async_remote_copy(
                    src_ref=vstage.at[hg], dst_ref=vg.at[p],
                    send_sem=vsend_sems.at[hg], recv_sem=vrecv_sems.at[p],
                    device_id=(hg,), device_id_type=pl.DeviceIdType.MESH,
                )
                rv.wait_send()

        for d in range(1, N_DEV):
            q = lax.rem(p + d, N_DEV)
            rp = pltpu.make_async_remote_copy(
                src_ref=prec.at[p], dst_ref=prec.at[p],
                send_sem=psend_sems.at[q], recv_sem=precv_sems.at[p],
                device_id=(q,), device_id_type=pl.DeviceIdType.MESH,
            )
            rp.wait_send()

    out_shape = jax.ShapeDtypeStruct((B, SQ, DMODEL), jnp.float32)
    return pl.pallas_call(
        body,
        out_shape=out_shape,
        in_specs=[pl.BlockSpec(memory_space=pltpu.VMEM)] * 5,
        out_specs=pl.BlockSpec(memory_space=pltpu.VMEM),
        scratch_shapes=[
            pltpu.VMEM((N_DEV, B, KV_SH, H_SH, DH), jnp.bfloat16),
            pltpu.VMEM((N_DEV, B, KV_SH, H_SH, DH), jnp.bfloat16),
            pltpu.VMEM((N_DEV, B, KV_SH, H_SH, DH), jnp.bfloat16),
            pltpu.VMEM((N_DEV, B, KV_SH, H_SH, DH), jnp.bfloat16),
            pltpu.VMEM((N_DEV, B, SQ, DMODEL), jnp.bfloat16),
            pltpu.SemaphoreType.DMA((N_DEV,)),
            pltpu.SemaphoreType.DMA((N_DEV,)),
            pltpu.SemaphoreType.DMA((N_DEV,)),
            pltpu.SemaphoreType.DMA((N_DEV,)),
            pltpu.SemaphoreType.DMA((N_DEV,)),
            pltpu.SemaphoreType.DMA((N_DEV,)),
        ],
        compiler_params=pltpu.CompilerParams(collective_id=0),
    )(x, Wq, K_ext, V_ext, Wo)


# device time: 33569 ns/iter; 1.0733x vs baseline; 1.0733x over previous
import jax
import jax.numpy as jnp
from jax import lax
from jax.experimental import pallas as pl
from jax.experimental.pallas import tpu as pltpu

N_DEV = 4
B, SQ, SKV, HQ, DH = 2, 128, 512, 16, 64
H_SH = HQ // N_DEV
KV_SH = SKV // N_DEV
DMODEL = 512
SL = DMODEL // N_DEV
SCALE = 0.125


def kernel(x, Wq, K_ext, V_ext, Wo):
    def body(x_ref, wq_ref, k_ref, v_ref, wo_ref, out_ref,
             kstage, vstage, kg, vg, pbuf, rs_buf, reduced, agstage, ag_buf,
             ksend_sems, vsend_sems, krecv_sems, vrecv_sems,
             rssend_sems, rsrecv_sems, agsend_sems, agrecv_sems):
        p = lax.axis_index("i")

        bar = pltpu.get_barrier_semaphore()
        for d in range(1, N_DEV):
            pl.semaphore_signal(
                bar, inc=1,
                device_id=(lax.rem(p + d, N_DEV),),
                device_id_type=pl.DeviceIdType.MESH,
            )
        pl.semaphore_wait(bar, N_DEV - 1)

        for hg in range(N_DEV):
            k_slice = k_ref[:, :, hg * H_SH:(hg + 1) * H_SH, :].astype(jnp.bfloat16)
            v_slice = v_ref[:, :, hg * H_SH:(hg + 1) * H_SH, :].astype(jnp.bfloat16)

            @pl.when(hg == p)
            def _(k_slice=k_slice, v_slice=v_slice):
                kg[hg] = k_slice
                vg[hg] = v_slice

            @pl.when(hg != p)
            def _(hg=hg, k_slice=k_slice, v_slice=v_slice):
                kstage[hg] = k_slice
                vstage[hg] = v_slice
                rk = pltpu.make_async_remote_copy(
                    src_ref=kstage.at[hg], dst_ref=kg.at[p],
                    send_sem=ksend_sems.at[hg], recv_sem=krecv_sems.at[p],
                    device_id=(hg,), device_id_type=pl.DeviceIdType.MESH,
                )
                rk.start()
                rv = pltpu.make_async_remote_copy(
                    src_ref=vstage.at[hg], dst_ref=vg.at[p],
                    send_sem=vsend_sems.at[hg], recv_sem=vrecv_sems.at[p],
                    device_id=(hg,), device_id_type=pl.DeviceIdType.MESH,
                )
                rv.start()

        wq = wq_ref[:, :].astype(jnp.bfloat16)
        q_b = []
        for b in range(B):
            qb = lax.dot_general(
                x_ref[b].astype(jnp.bfloat16), wq,
                (((1,), (0,)), ((), ())),
                preferred_element_type=jnp.float32,
            )
            q_b.append(qb.astype(jnp.bfloat16))

        qi = lax.broadcasted_iota(jnp.int32, (SQ, SKV), 0)
        ki = lax.broadcasted_iota(jnp.int32, (SQ, SKV), 1)
        mask = (ki <= qi + 128) | (qi < 32)

        for d in range(1, N_DEV):
            o = lax.rem(p + d, N_DEV)
            rk = pltpu.make_async_remote_copy(
                src_ref=kg.at[o], dst_ref=kg.at[o],
                send_sem=ksend_sems.at[0], recv_sem=krecv_sems.at[o],
                device_id=(o,), device_id_type=pl.DeviceIdType.MESH,
            )
            rk.wait_recv()
            rv = pltpu.make_async_remote_copy(
                src_ref=vg.at[o], dst_ref=vg.at[o],
                send_sem=vsend_sems.at[0], recv_sem=vrecv_sems.at[o],
                device_id=(o,), device_id_type=pl.DeviceIdType.MESH,
            )
            rv.wait_recv()

        wo = wo_ref[:, :].astype(jnp.bfloat16)
        for b in range(B):
            ctx_h = []
            for h in range(H_SH):
                q_bh = q_b[b][:, h * DH:(h + 1) * DH]
                k_full = jnp.concatenate(
                    [kg[o, b, :, h, :] for o in range(N_DEV)], axis=0
                )
                s = lax.dot_general(
                    q_bh, k_full, (((1,), (1,)), ((), ())),
                    preferred_element_type=jnp.float32,
                ) * SCALE
                ef = jnp.where(mask, jnp.exp(s), 0.0)
                den = jnp.sum(ef, axis=1, keepdims=True)
                v_full = jnp.concatenate(
                    [vg[o, b, :, h, :] for o in range(N_DEV)], axis=0
                )
                ctx_un = lax.dot_general(
                    ef.astype(jnp.bfloat16), v_full,
                    (((1,), (0,)), ((), ())),
                    preferred_element_type=jnp.float32,
                )
                ctx_h.append(ctx_un * (1.0 / den))
            ctx_b = jnp.concatenate(ctx_h, axis=1).astype(jnp.bfloat16)
            partial_b = lax.dot_general(
                ctx_b, wo, (((1,), (0,)), ((), ())),
                preferred_element_type=jnp.float32,
            )
            pbuf[b] = partial_b.astype(jnp.bfloat16)
            for hg in range(N_DEV):
                @pl.when(hg == p)
                def _(hg=hg, partial_b=partial_b, b=b):
                    reduced[b] = partial_b[:, hg * SL:(hg + 1) * SL]
            for dest in range(N_DEV):
                @pl.when(dest != p)
                def _(dest=dest, b=b):
                    r = pltpu.make_async_remote_copy(
                        src_ref=pbuf.at[b, :, pl.ds(dest * SL, SL)],
                        dst_ref=rs_buf.at[p, b],
                        send_sem=rssend_sems.at[dest, b],
                        recv_sem=rsrecv_sems.at[p, b],
                        device_id=(dest,), device_id_type=pl.DeviceIdType.MESH,
                    )
                    r.start()

        for b in range(B):
            for d in range(1, N_DEV):
                o = lax.rem(p + d, N_DEV)
                r = pltpu.make_async_remote_copy(
                    src_ref=rs_buf.at[o, b], dst_ref=rs_buf.at[o, b],
                    send_sem=rssend_sems.at[0, b], recv_sem=rsrecv_sems.at[o, b],
                    device_id=(o,), device_id_type=pl.DeviceIdType.MESH,
                )
                r.wait_recv()
            acc = reduced[b]
            for d in range(1, N_DEV):
                o = lax.rem(p + d, N_DEV)
                acc = acc + rs_buf[o, b].astype(jnp.float32)
            reduced[b] = acc
            agstage[b] = acc.astype(jnp.bfloat16)

        for dest in range(N_DEV):
            @pl.when(dest != p)
            def _(dest=dest):
                r = pltpu.make_async_remote_copy(
                    src_ref=agstage, dst_ref=ag_buf.at[p],
                    send_sem=agsend_sems.at[dest], recv_sem=agrecv_sems.at[p],
                    device_id=(dest,), device_id_type=pl.DeviceIdType.MESH,
                )
                r.start()

        for hg in range(N_DEV):
            @pl.when(hg == p)
            def _(hg=hg):
                out_ref[:, :, hg * SL:(hg + 1) * SL] = reduced[:, :, :]

        for d in range(1, N_DEV):
            o = lax.rem(p + d, N_DEV)
            r = pltpu.make_async_remote_copy(
                src_ref=ag_buf.at[o], dst_ref=ag_buf.at[o],
                send_sem=agsend_sems.at[0], recv_sem=agrecv_sems.at[o],
                device_id=(o,), device_id_type=pl.DeviceIdType.MESH,
            )
            r.wait_recv()
        for hg in range(N_DEV):
            @pl.when(hg != p)
            def _(hg=hg):
                out_ref[:, :, hg * SL:(hg + 1) * SL] = ag_buf[hg].astype(jnp.float32)

        for hg in range(N_DEV):
            @pl.when(hg != p)
            def _(hg=hg):
                rk = pltpu.make_async_remote_copy(
                    src_ref=kstage.at[hg], dst_ref=kg.at[p],
                    send_sem=ksend_sems.at[hg], recv_sem=krecv_sems.at[p],
                    device_id=(hg,), device_id_type=pl.DeviceIdType.MESH,
                )
                rk.wait_send()
                rv = pltpu.make_async_remote_copy(
                    src_ref=vstage.at[hg], dst_ref=vg.at[p],
                    send_sem=vsend_sems.at[hg], recv_sem=vrecv_sems.at[p],
                    device_id=(hg,), device_id_type=pl.DeviceIdType.MESH,
                )
                rv.wait_send()
                for b in range(B):
                    r = pltpu.make_async_remote_copy(
                        src_ref=pbuf.at[b, :, pl.ds(hg * SL, SL)],
                        dst_ref=rs_buf.at[p, b],
                        send_sem=rssend_sems.at[hg, b],
                        recv_sem=rsrecv_sems.at[p, b],
                        device_id=(hg,), device_id_type=pl.DeviceIdType.MESH,
                    )
                    r.wait_send()
                ra = pltpu.make_async_remote_copy(
                    src_ref=agstage, dst_ref=ag_buf.at[p],
                    send_sem=agsend_sems.at[hg], recv_sem=agrecv_sems.at[p],
                    device_id=(hg,), device_id_type=pl.DeviceIdType.MESH,
                )
                ra.wait_send()

    out_shape = jax.ShapeDtypeStruct((B, SQ, DMODEL), jnp.float32)
    return pl.pallas_call(
        body,
        out_shape=out_shape,
        in_specs=[pl.BlockSpec(memory_space=pltpu.VMEM)] * 5,
        out_specs=pl.BlockSpec(memory_space=pltpu.VMEM),
        scratch_shapes=[
            pltpu.VMEM((N_DEV, B, KV_SH, H_SH, DH), jnp.bfloat16),
            pltpu.VMEM((N_DEV, B, KV_SH, H_SH, DH), jnp.bfloat16),
            pltpu.VMEM((N_DEV, B, KV_SH, H_SH, DH), jnp.bfloat16),
            pltpu.VMEM((N_DEV, B, KV_SH, H_SH, DH), jnp.bfloat16),
            pltpu.VMEM((B, SQ, DMODEL), jnp.bfloat16),
            pltpu.VMEM((N_DEV, B, SQ, SL), jnp.bfloat16),
            pltpu.VMEM((B, SQ, SL), jnp.float32),
            pltpu.VMEM((B, SQ, SL), jnp.bfloat16),
            pltpu.VMEM((N_DEV, B, SQ, SL), jnp.bfloat16),
            pltpu.SemaphoreType.DMA((N_DEV,)),
            pltpu.SemaphoreType.DMA((N_DEV,)),
            pltpu.SemaphoreType.DMA((N_DEV,)),
            pltpu.SemaphoreType.DMA((N_DEV,)),
            pltpu.SemaphoreType.DMA((N_DEV, B)),
            pltpu.SemaphoreType.DMA((N_DEV, B)),
            pltpu.SemaphoreType.DMA((N_DEV,)),
            pltpu.SemaphoreType.DMA((N_DEV,)),
        ],
        compiler_params=pltpu.CompilerParams(collective_id=0),
    )(x, Wq, K_ext, V_ext, Wo)


# device time: 24497 ns/iter; 1.4708x vs baseline; 1.3703x over previous
import jax
import jax.numpy as jnp
from jax import lax
from jax.experimental import pallas as pl
from jax.experimental.pallas import tpu as pltpu

N_DEV = 4
B, SQ, SKV, HQ, DH = 2, 128, 512, 16, 64
H_SH = HQ // N_DEV
KV_SH = SKV // N_DEV
DMODEL = 512
SL = DMODEL // N_DEV
SCALE = 0.125


def kernel(x, Wq, K_ext, V_ext, Wo):
    kv = jnp.stack([K_ext, V_ext])
    scales = jnp.max(jnp.abs(kv), axis=(1, 2, 3, 4)) / 127.0
    kvt = jnp.transpose(
        jnp.clip(jnp.round(kv / scales[:, None, None, None, None]), -127, 127),
        (0, 3, 1, 2, 4),
    ).astype(jnp.int8)
    scl = jnp.broadcast_to(scales[:, None], (2, 128)).astype(jnp.float32)

    def body(x_ref, wq_ref, kvt_ref, wo_ref, scl_ref, out_ref,
             kg, vg, sclg, pbuf, rs_buf, reduced, agstage, ag_buf,
             ksend_sems, vsend_sems, sclsend_sems,
             krecv_sems, vrecv_sems, sclrecv_sems,
             rssend_sems, rsrecv_sems, agsend_sems, agrecv_sems):
        p = lax.axis_index("i")

        bar = pltpu.get_barrier_semaphore()
        for d in range(1, N_DEV):
            pl.semaphore_signal(
                bar, inc=1,
                device_id=(lax.rem(p + d, N_DEV),),
                device_id_type=pl.DeviceIdType.MESH,
            )
        pl.semaphore_wait(bar, N_DEV - 1)

        for t in range(1, N_DEV):
            dest = lax.rem(p + t, N_DEV)
            rk = pltpu.make_async_remote_copy(
                src_ref=kvt_ref.at[0, pl.ds(dest * H_SH, H_SH)],
                dst_ref=kg.at[t],
                send_sem=ksend_sems.at[t], recv_sem=krecv_sems.at[t],
                device_id=(dest,), device_id_type=pl.DeviceIdType.MESH,
            )
            rk.start()
            rv = pltpu.make_async_remote_copy(
                src_ref=kvt_ref.at[1, pl.ds(dest * H_SH, H_SH)],
                dst_ref=vg.at[t],
                send_sem=vsend_sems.at[t], recv_sem=vrecv_sems.at[t],
                device_id=(dest,), device_id_type=pl.DeviceIdType.MESH,
            )
            rv.start()
            rs = pltpu.make_async_remote_copy(
                src_ref=scl_ref,
                dst_ref=sclg.at[t],
                send_sem=sclsend_sems.at[t], recv_sem=sclrecv_sems.at[t],
                device_id=(dest,), device_id_type=pl.DeviceIdType.MESH,
            )
            rs.start()

        for hg in range(N_DEV):
            @pl.when(hg == p)
            def _(hg=hg):
                kg[0] = kvt_ref[0, hg * H_SH:(hg + 1) * H_SH]
                vg[0] = kvt_ref[1, hg * H_SH:(hg + 1) * H_SH]

        wq = wq_ref[:, :].astype(jnp.bfloat16)
        q_b = []
        for b in range(B):
            qb = lax.dot_general(
                x_ref[b].astype(jnp.bfloat16), wq, (((1,), (0,)), ((), ())),
                preferred_element_type=jnp.float32,
            )
            q_b.append(qb.astype(jnp.bfloat16))

        qi = lax.broadcasted_iota(jnp.int32, (SQ, KV_SH), 0)
        ki = lax.broadcasted_iota(jnp.int32, (SQ, KV_SH), 1)
        qlt32 = qi < 32

        ctx = [[None] * H_SH for _ in range(B)]
        den = [[None] * H_SH for _ in range(B)]

        def compute_scores(slot, origin, kscale):
            m = (ki + origin * KV_SH <= qi + 128) | qlt32
            sk = kscale * SCALE
            efs = []
            for b in range(B):
                for h in range(H_SH):
                    q_bh = q_b[b][:, h * DH:(h + 1) * DH]
                    s = lax.dot_general(
                        q_bh, kg[slot, h, b].astype(jnp.bfloat16),
                        (((1,), (1,)), ((), ())),
                        preferred_element_type=jnp.float32,
                    ) * sk
                    ef = jnp.where(m, jnp.exp(s), 0.0)
                    d_c = jnp.sum(ef, axis=1, keepdims=True)
                    if den[b][h] is None:
                        den[b][h] = d_c
                    else:
                        den[b][h] = den[b][h] + d_c
                    efs.append(ef)
            return efs

        def apply_pv(slot, efs, vscale):
            for b in range(B):
                for h in range(H_SH):
                    c_c = lax.dot_general(
                        (efs[b * H_SH + h] * vscale).astype(jnp.bfloat16),
                        vg[slot, h, b].astype(jnp.bfloat16),
                        (((1,), (0,)), ((), ())),
                        preferred_element_type=jnp.float32,
                    )
                    if ctx[b][h] is None:
                        ctx[b][h] = c_c
                    else:
                        ctx[b][h] = ctx[b][h] + c_c

        efs0 = compute_scores(0, p, scl_ref[0, 0])
        apply_pv(0, efs0, scl_ref[1, 0])

        for t in range(1, N_DEV):
            o = lax.rem(p + (N_DEV - t), N_DEV)
            rs = pltpu.make_async_remote_copy(
                src_ref=sclg.at[t], dst_ref=sclg.at[t],
                send_sem=sclsend_sems.at[t], recv_sem=sclrecv_sems.at[t],
                device_id=(o,), device_id_type=pl.DeviceIdType.MESH,
            )
            rs.wait_recv()
            rk = pltpu.make_async_remote_copy(
                src_ref=kg.at[t], dst_ref=kg.at[t],
                send_sem=ksend_sems.at[t], recv_sem=krecv_sems.at[t],
                device_id=(o,), device_id_type=pl.DeviceIdType.MESH,
            )
            rk.wait_recv()
            efs = compute_scores(t, o, sclg[t, 0, 0])
            rv = pltpu.make_async_remote_copy(
                src_ref=vg.at[t], dst_ref=vg.at[t],
                send_sem=vsend_sems.at[t], recv_sem=vrecv_sems.at[t],
                device_id=(o,), device_id_type=pl.DeviceIdType.MESH,
            )
            rv.wait_recv()
            apply_pv(t, efs, sclg[t, 1, 0])

        wo = wo_ref[:, :].astype(jnp.bfloat16)
        for b in range(B):
            ctx_b = jnp.concatenate(
                [ctx[b][h] * (1.0 / den[b][h]) for h in range(H_SH)], axis=1
            ).astype(jnp.bfloat16)
            partial_b = lax.dot_general(
                ctx_b, wo, (((1,), (0,)), ((), ())),
                preferred_element_type=jnp.float32,
            )
            pbuf[b] = partial_b.astype(jnp.bfloat16)
            for hg in range(N_DEV):
                @pl.when(hg == p)
                def _(hg=hg, partial_b=partial_b, b=b):
                    reduced[b] = partial_b[:, hg * SL:(hg + 1) * SL]
            for t in range(1, N_DEV):
                dest = lax.rem(p + t, N_DEV)
                r = pltpu.make_async_remote_copy(
                    src_ref=pbuf.at[b, :, pl.ds(dest * SL, SL)],
                    dst_ref=rs_buf.at[p, b],
                    send_sem=rssend_sems.at[t, b],
                    recv_sem=rsrecv_sems.at[p, b],
                    device_id=(dest,), device_id_type=pl.DeviceIdType.MESH,
                )
                r.start()

        for b in range(B):
            for t in range(1, N_DEV):
                o = lax.rem(p + (N_DEV - t), N_DEV)
                r = pltpu.make_async_remote_copy(
                    src_ref=rs_buf.at[o, b], dst_ref=rs_buf.at[o, b],
                    send_sem=rssend_sems.at[t, b], recv_sem=rsrecv_sems.at[o, b],
                    device_id=(o,), device_id_type=pl.DeviceIdType.MESH,
                )
                r.wait_recv()
            acc = reduced[b]
            for d in range(1, N_DEV):
                o = lax.rem(p + d, N_DEV)
                acc = acc + rs_buf[o, b].astype(jnp.float32)
            agstage[b] = acc.astype(jnp.bfloat16)
            for hg in range(N_DEV):
                @pl.when(hg == p)
                def _(hg=hg, acc=acc, b=b):
                    out_ref[b, :, hg * SL:(hg + 1) * SL] = acc
            for t in range(1, N_DEV):
                dest = lax.rem(p + t, N_DEV)
                r = pltpu.make_async_remote_copy(
                    src_ref=agstage.at[b], dst_ref=ag_buf.at[p, b],
                    send_sem=agsend_sems.at[t, b],
                    recv_sem=agrecv_sems.at[p, b],
                    device_id=(dest,), device_id_type=pl.DeviceIdType.MESH,
                )
                r.start()

        for t in range(1, N_DEV):
            o = lax.rem(p + (N_DEV - t), N_DEV)
            for b in range(B):
                r = pltpu.make_async_remote_copy(
                    src_ref=ag_buf.at[o, b], dst_ref=ag_buf.at[o, b],
                    send_sem=agsend_sems.at[t, b], recv_sem=agrecv_sems.at[o, b],
                    device_id=(o,), device_id_type=pl.DeviceIdType.MESH,
                )
                r.wait_recv()
        for hg in range(N_DEV):
            @pl.when(hg != p)
            def _(hg=hg):
                out_ref[:, :, hg * SL:(hg + 1) * SL] = ag_buf[hg].astype(jnp.float32)

        for t in range(1, N_DEV):
            dest = lax.rem(p + t, N_DEV)
            rk = pltpu.make_async_remote_copy(
                src_ref=kvt_ref.at[0, pl.ds(dest * H_SH, H_SH)],
                dst_ref=kg.at[t],
                send_sem=ksend_sems.at[t], recv_sem=krecv_sems.at[t],
                device_id=(dest,), device_id_type=pl.DeviceIdType.MESH,
            )
            rk.wait_send()
            rv = pltpu.make_async_remote_copy(
                src_ref=kvt_ref.at[1, pl.ds(dest * H_SH, H_SH)],
                dst_ref=vg.at[t],
                send_sem=vsend_sems.at[t], recv_sem=vrecv_sems.at[t],
                device_id=(dest,), device_id_type=pl.DeviceIdType.MESH,
            )
            rv.wait_send()
            rsc = pltpu.make_async_remote_copy(
                src_ref=scl_ref, dst_ref=sclg.at[t],
                send_sem=sclsend_sems.at[t], recv_sem=sclrecv_sems.at[t],
                device_id=(dest,), device_id_type=pl.DeviceIdType.MESH,
            )
            rsc.wait_send()
            for b in range(B):
                r = pltpu.make_async_remote_copy(
                    src_ref=pbuf.at[b, :, pl.ds(dest * SL, SL)],
                    dst_ref=rs_buf.at[p, b],
                    send_sem=rssend_sems.at[t, b], recv_sem=rsrecv_sems.at[p, b],
                    device_id=(dest,), device_id_type=pl.DeviceIdType.MESH,
                )
                r.wait_send()
                ra = pltpu.make_async_remote_copy(
                    src_ref=agstage.at[b], dst_ref=ag_buf.at[p, b],
                    send_sem=agsend_sems.at[t, b], recv_sem=agrecv_sems.at[p, b],
                    device_id=(dest,), device_id_type=pl.DeviceIdType.MESH,
                )
                ra.wait_send()

    out_shape = jax.ShapeDtypeStruct((B, SQ, DMODEL), jnp.float32)
    return pl.pallas_call(
        body,
        out_shape=out_shape,
        in_specs=[pl.BlockSpec(memory_space=pltpu.VMEM)] * 5,
        out_specs=pl.BlockSpec(memory_space=pltpu.VMEM),
        scratch_shapes=[
            pltpu.VMEM((N_DEV, H_SH, B, KV_SH, DH), jnp.int8),
            pltpu.VMEM((N_DEV, H_SH, B, KV_SH, DH), jnp.int8),
            pltpu.VMEM((N_DEV, 2, 128), jnp.float32),
            pltpu.VMEM((B, SQ, DMODEL), jnp.bfloat16),
            pltpu.VMEM((N_DEV, B, SQ, SL), jnp.bfloat16),
            pltpu.VMEM((B, SQ, SL), jnp.float32),
            pltpu.VMEM((B, SQ, SL), jnp.bfloat16),
            pltpu.VMEM((N_DEV, B, SQ, SL), jnp.bfloat16),
            pltpu.SemaphoreType.DMA((N_DEV,)),
            pltpu.SemaphoreType.DMA((N_DEV,)),
            pltpu.SemaphoreType.DMA((N_DEV,)),
            pltpu.SemaphoreType.DMA((N_DEV,)),
            pltpu.SemaphoreType.DMA((N_DEV,)),
            pltpu.SemaphoreType.DMA((N_DEV,)),
            pltpu.SemaphoreType.DMA((N_DEV, B)),
            pltpu.SemaphoreType.DMA((N_DEV, B)),
            pltpu.SemaphoreType.DMA((N_DEV, B)),
            pltpu.SemaphoreType.DMA((N_DEV, B)),
        ],
        compiler_params=pltpu.CompilerParams(collective_id=0),
    )(x, Wq, kvt, Wo, scl)
